# Initial kernel scaffold; baseline (speedup 1.0000x reference)
import jax
import jax.numpy as jnp
from jax.experimental import pallas as pl


def kernel(x, stn_g1_Wr, stn_g1_Wn, stn_g1_b, stn_g2_Wr, stn_g2_Wn, stn_g2_b, stn_g3_Wr, stn_g3_Wn, stn_g3_b, stn_fc1_W, stn_fc1_b, stn_fc2_W, stn_fc2_b, stn_fc3_W, stn_fc3_b, c1_Wr, c1_Wn, c1_b, c2_Wr, c2_Wn, c2_b, c3_Wr, c3_Wn, c3_b):
    raise NotImplementedError("write your pallas kernel here")


# speedup vs baseline: 3.0393x; 3.0393x over previous
import jax
import jax.numpy as jnp
from jax.experimental import pallas as pl
from jax.experimental.pallas import tpu as pltpu

_BLOCK = 2000
_NEG = -jnp.inf


def _fused_kernel(x_ref,
                  sWr1, sWn1, sb1, sWr2, sWn2, sb2, sWr3, sWn3, sb3,
                  fc1W, fc1b, fc2W, fc2b, fc3W, fc3b,
                  cWr1, cWn1, cb1, cWr2, cWn2, cb2, cWr3, cWn3, cb3,
                  h_out, t9_out,
                  smax, cmax):
    i = pl.program_id(0)
    nsteps = pl.num_programs(0)
    xb = x_ref[...]

    def mm(a, w):
        return jax.lax.dot_general(a, w, (((1,), (0,)), ((), ())),
                                   preferred_element_type=jnp.float32)

    hs = jnp.maximum(mm(xb, sWr1[...]) + sb1[...], 0.0)
    hs = jnp.maximum(mm(hs, sWr2[...]) + sb2[...], 0.0)
    hs = jnp.maximum(mm(hs, sWr3[...]) + sb3[...], 0.0)
    hc = jnp.maximum(mm(xb, cWr1[...]) + cb1[...], 0.0)
    hc = jnp.maximum(mm(hc, cWr2[...]) + cb2[...], 0.0)
    hc = mm(hc, cWr3[...]) + cb3[...]

    @pl.when(i == 0)
    def _init():
        rows = jax.lax.broadcasted_iota(jnp.int32, (_BLOCK, 1), 0)
        bad = rows == 1
        bs = jnp.max(jnp.where(bad, _NEG, hs), axis=0, keepdims=True)
        bc = jnp.max(jnp.where(bad, _NEG, hc), axis=0, keepdims=True)

        x2 = xb[0:2, :]
        sel = (jax.lax.broadcasted_iota(jnp.int32, (2, 1), 0) == 1
               ).astype(jnp.float32)

        def gconv(h2, wr, wn, b):
            return mm(h2, wr[...]) + b[...] + sel * mm(h2[0:1, :], wn[...])

        e = jnp.maximum(gconv(x2, sWr1, sWn1, sb1), 0.0)
        e = jnp.maximum(gconv(e, sWr2, sWn2, sb2), 0.0)
        e = jnp.maximum(gconv(e, sWr3, sWn3, sb3), 0.0)
        es = jnp.max(e, axis=0, keepdims=True)
        e = jnp.maximum(gconv(x2, cWr1, cWn1, cb1), 0.0)
        e = jnp.maximum(gconv(e, cWr2, cWn2, cb2), 0.0)
        e = gconv(e, cWr3, cWn3, cb3)
        ec = jnp.max(e, axis=0, keepdims=True)

        smax[...] = jnp.maximum(bs, es)
        cmax[...] = jnp.maximum(bc, ec)

    @pl.when(i > 0)
    def _acc():
        smax[...] = jnp.maximum(smax[...], jnp.max(hs, axis=0, keepdims=True))
        cmax[...] = jnp.maximum(cmax[...], jnp.max(hc, axis=0, keepdims=True))

    @pl.when(i == nsteps - 1)
    def _tail():
        h_out[...] = cmax[...]
        t = jnp.maximum(mm(smax[...], fc1W[...]) + fc1b[...], 0.0)
        t = jnp.maximum(mm(t, fc2W[...]) + fc2b[...], 0.0)
        t9 = mm(t, fc3W[...]) + fc3b[...]
        col = jax.lax.broadcasted_iota(jnp.int32, (1, 9), 1)
        t9_out[...] = t9 + (col % 4 == 0).astype(jnp.float32)


def kernel(x, stn_g1_Wr, stn_g1_Wn, stn_g1_b, stn_g2_Wr, stn_g2_Wn, stn_g2_b,
           stn_g3_Wr, stn_g3_Wn, stn_g3_b, stn_fc1_W, stn_fc1_b,
           stn_fc2_W, stn_fc2_b, stn_fc3_W, stn_fc3_b,
           c1_Wr, c1_Wn, c1_b, c2_Wr, c2_Wn, c2_b, c3_Wr, c3_Wn, c3_b):
    n = x.shape[0]
    grid = n // _BLOCK
    assert grid * _BLOCK == n

    row = lambda v: v.reshape(1, -1)
    weights = (
        stn_g1_Wr, stn_g1_Wn, row(stn_g1_b),
        stn_g2_Wr, stn_g2_Wn, row(stn_g2_b),
        stn_g3_Wr, stn_g3_Wn, row(stn_g3_b),
        stn_fc1_W, row(stn_fc1_b), stn_fc2_W, row(stn_fc2_b),
        stn_fc3_W, row(stn_fc3_b),
        c1_Wr, c1_Wn, row(c1_b),
        c2_Wr, c2_Wn, row(c2_b),
        c3_Wr, c3_Wn, row(c3_b),
    )
    wspecs = [pl.BlockSpec(w.shape, lambda i: (0, 0)) for w in weights]

    h, t9 = pl.pallas_call(
        _fused_kernel,
        grid=(grid,),
        in_specs=[pl.BlockSpec((_BLOCK, 3), lambda i: (i, 0))] + wspecs,
        out_specs=[pl.BlockSpec((1, 1024), lambda i: (0, 0)),
                   pl.BlockSpec((1, 9), lambda i: (0, 0))],
        out_shape=[jax.ShapeDtypeStruct((1, 1024), jnp.float32),
                   jax.ShapeDtypeStruct((1, 9), jnp.float32)],
        scratch_shapes=[pltpu.VMEM((1, 1024), jnp.float32),
                        pltpu.VMEM((1, 1024), jnp.float32)],
    )(x, *weights)
    return h, t9.reshape(3, 3)


# speedup vs baseline: 3.2861x; 1.0812x over previous
import jax
import jax.numpy as jnp
from jax.experimental import pallas as pl
from jax.experimental.pallas import tpu as pltpu

_BLOCK = 4000
_NEG = -jnp.inf


def _fused_kernel(x_ref,
                  sWr1, sWn1, sb1, sWr2, sWn2, sb2, sWr3, sWn3, sb3,
                  fc1W, fc1b, fc2W, fc2b, fc3W, fc3b,
                  cWr1, cWn1, cb1, cWr2, cWn2, cb2, cWr3, cWn3, cb3,
                  h_out, t9_out,
                  smax, cmax):
    i = pl.program_id(0)
    nsteps = pl.num_programs(0)
    xb = x_ref[...]

    def mm(a, w):
        return jax.lax.dot_general(a, w, (((1,), (0,)), ((), ())),
                                   preferred_element_type=jnp.float32)

    hs = jnp.maximum(mm(xb, sWr1[...]) + sb1[...], 0.0)
    hs = jnp.maximum(mm(hs, sWr2[...]) + sb2[...], 0.0)
    hs = mm(hs, sWr3[...])
    hc = jnp.maximum(mm(xb, cWr1[...]) + cb1[...], 0.0)
    hc = jnp.maximum(mm(hc, cWr2[...]) + cb2[...], 0.0)
    hc = mm(hc, cWr3[...])

    @pl.when(i == 0)
    def _init():
        rows = jax.lax.broadcasted_iota(jnp.int32, (_BLOCK, 1), 0)
        bad = rows == 1
        bs = jnp.max(jnp.where(bad, _NEG, hs), axis=0, keepdims=True)
        bc = jnp.max(jnp.where(bad, _NEG, hc), axis=0, keepdims=True)

        x2 = xb[0:2, :]
        sel = (jax.lax.broadcasted_iota(jnp.int32, (2, 1), 0) == 1
               ).astype(jnp.float32)

        def gconv(h2, wr, wn):
            return mm(h2, wr[...]) + sel * mm(h2[0:1, :], wn[...])

        e = jnp.maximum(gconv(x2, sWr1, sWn1) + sb1[...], 0.0)
        e = jnp.maximum(gconv(e, sWr2, sWn2) + sb2[...], 0.0)
        e = gconv(e, sWr3, sWn3)
        es = jnp.max(e, axis=0, keepdims=True)
        e = jnp.maximum(gconv(x2, cWr1, cWn1) + cb1[...], 0.0)
        e = jnp.maximum(gconv(e, cWr2, cWn2) + cb2[...], 0.0)
        e = gconv(e, cWr3, cWn3)
        ec = jnp.max(e, axis=0, keepdims=True)

        smax[...] = jnp.maximum(bs, es)
        cmax[...] = jnp.maximum(bc, ec)

    @pl.when(i > 0)
    def _acc():
        smax[...] = jnp.maximum(smax[...], jnp.max(hs, axis=0, keepdims=True))
        cmax[...] = jnp.maximum(cmax[...], jnp.max(hc, axis=0, keepdims=True))

    @pl.when(i == nsteps - 1)
    def _tail():
        h_out[...] = cmax[...] + cb3[...]
        s = jnp.maximum(smax[...] + sb3[...], 0.0)
        t = jnp.maximum(mm(s, fc1W[...]) + fc1b[...], 0.0)
        t = jnp.maximum(mm(t, fc2W[...]) + fc2b[...], 0.0)
        t9 = mm(t, fc3W[...]) + fc3b[...]
        col = jax.lax.broadcasted_iota(jnp.int32, (1, 9), 1)
        t9_out[...] = t9 + (col % 4 == 0).astype(jnp.float32)


def kernel(x, stn_g1_Wr, stn_g1_Wn, stn_g1_b, stn_g2_Wr, stn_g2_Wn, stn_g2_b,
           stn_g3_Wr, stn_g3_Wn, stn_g3_b, stn_fc1_W, stn_fc1_b,
           stn_fc2_W, stn_fc2_b, stn_fc3_W, stn_fc3_b,
           c1_Wr, c1_Wn, c1_b, c2_Wr, c2_Wn, c2_b, c3_Wr, c3_Wn, c3_b):
    n = x.shape[0]
    grid = n // _BLOCK
    assert grid * _BLOCK == n

    row = lambda v: v.reshape(1, -1)
    weights = (
        stn_g1_Wr, stn_g1_Wn, row(stn_g1_b),
        stn_g2_Wr, stn_g2_Wn, row(stn_g2_b),
        stn_g3_Wr, stn_g3_Wn, row(stn_g3_b),
        stn_fc1_W, row(stn_fc1_b), stn_fc2_W, row(stn_fc2_b),
        stn_fc3_W, row(stn_fc3_b),
        c1_Wr, c1_Wn, row(c1_b),
        c2_Wr, c2_Wn, row(c2_b),
        c3_Wr, c3_Wn, row(c3_b),
    )
    wspecs = [pl.BlockSpec(w.shape, lambda i: (0, 0)) for w in weights]

    h, t9 = pl.pallas_call(
        _fused_kernel,
        grid=(grid,),
        in_specs=[pl.BlockSpec((_BLOCK, 3), lambda i: (i, 0))] + wspecs,
        out_specs=[pl.BlockSpec((1, 1024), lambda i: (0, 0)),
                   pl.BlockSpec((1, 9), lambda i: (0, 0))],
        out_shape=[jax.ShapeDtypeStruct((1, 1024), jnp.float32),
                   jax.ShapeDtypeStruct((1, 9), jnp.float32)],
        scratch_shapes=[pltpu.VMEM((1, 1024), jnp.float32),
                        pltpu.VMEM((1, 1024), jnp.float32)],
    )(x, *weights)
    return h, t9.reshape(3, 3)


# speedup vs baseline: 3.3315x; 1.0138x over previous
import jax
import jax.numpy as jnp
from jax.experimental import pallas as pl
from jax.experimental.pallas import tpu as pltpu

_BLOCK = 4000
_NEG = -jnp.inf


def _fused_kernel(x_ref,
                  sWr1, sWn1, sb1, sWr2, sWn2, sb2, sWr3, sWn3, sb3,
                  fc1W, fc1b, fc2W, fc2b, fc3W, fc3b,
                  cWr1, cWn1, cb1, cWr2, cWn2, cb2, cWr3, cWn3, cb3,
                  h_out, t9_out,
                  smax, cmax):
    i = pl.program_id(0)
    nsteps = pl.num_programs(0)
    xb = x_ref[...]

    def mm(a, w):
        return jax.lax.dot_general(a, w, (((1,), (0,)), ((), ())),
                                   preferred_element_type=jnp.float32)

    def mm16(a, w):
        return jax.lax.dot_general(a.astype(jnp.bfloat16),
                                   w.astype(jnp.bfloat16),
                                   (((1,), (0,)), ((), ())),
                                   preferred_element_type=jnp.float32)

    hs = jnp.maximum(mm(xb, sWr1[...]) + sb1[...], 0.0)
    hs = jnp.maximum(mm(hs, sWr2[...]) + sb2[...], 0.0)
    hs = mm16(hs, sWr3[...])
    hc = jnp.maximum(mm(xb, cWr1[...]) + cb1[...], 0.0)
    hc = jnp.maximum(mm(hc, cWr2[...]) + cb2[...], 0.0)
    hc = mm16(hc, cWr3[...])

    @pl.when(i == 0)
    def _init():
        rows = jax.lax.broadcasted_iota(jnp.int32, (_BLOCK, 1), 0)
        bad = rows == 1
        bs = jnp.max(jnp.where(bad, _NEG, hs), axis=0, keepdims=True)
        bc = jnp.max(jnp.where(bad, _NEG, hc), axis=0, keepdims=True)

        x2 = xb[0:2, :]
        sel = (jax.lax.broadcasted_iota(jnp.int32, (2, 1), 0) == 1
               ).astype(jnp.float32)

        def gconv(h2, wr, wn):
            return mm(h2, wr[...]) + sel * mm(h2[0:1, :], wn[...])

        e = jnp.maximum(gconv(x2, sWr1, sWn1) + sb1[...], 0.0)
        e = jnp.maximum(gconv(e, sWr2, sWn2) + sb2[...], 0.0)
        e = gconv(e, sWr3, sWn3)
        es = jnp.max(e, axis=0, keepdims=True)
        e = jnp.maximum(gconv(x2, cWr1, cWn1) + cb1[...], 0.0)
        e = jnp.maximum(gconv(e, cWr2, cWn2) + cb2[...], 0.0)
        e = gconv(e, cWr3, cWn3)
        ec = jnp.max(e, axis=0, keepdims=True)

        smax[...] = jnp.maximum(bs, es)
        cmax[...] = jnp.maximum(bc, ec)

    @pl.when(i > 0)
    def _acc():
        smax[...] = jnp.maximum(smax[...], jnp.max(hs, axis=0, keepdims=True))
        cmax[...] = jnp.maximum(cmax[...], jnp.max(hc, axis=0, keepdims=True))

    @pl.when(i == nsteps - 1)
    def _tail():
        h_out[...] = cmax[...] + cb3[...]
        s = jnp.maximum(smax[...] + sb3[...], 0.0)
        t = jnp.maximum(mm(s, fc1W[...]) + fc1b[...], 0.0)
        t = jnp.maximum(mm(t, fc2W[...]) + fc2b[...], 0.0)
        t9 = mm(t, fc3W[...]) + fc3b[...]
        col = jax.lax.broadcasted_iota(jnp.int32, (1, 9), 1)
        t9_out[...] = t9 + (col % 4 == 0).astype(jnp.float32)


def kernel(x, stn_g1_Wr, stn_g1_Wn, stn_g1_b, stn_g2_Wr, stn_g2_Wn, stn_g2_b,
           stn_g3_Wr, stn_g3_Wn, stn_g3_b, stn_fc1_W, stn_fc1_b,
           stn_fc2_W, stn_fc2_b, stn_fc3_W, stn_fc3_b,
           c1_Wr, c1_Wn, c1_b, c2_Wr, c2_Wn, c2_b, c3_Wr, c3_Wn, c3_b):
    n = x.shape[0]
    grid = n // _BLOCK
    assert grid * _BLOCK == n

    row = lambda v: v.reshape(1, -1)
    weights = (
        stn_g1_Wr, stn_g1_Wn, row(stn_g1_b),
        stn_g2_Wr, stn_g2_Wn, row(stn_g2_b),
        stn_g3_Wr, stn_g3_Wn, row(stn_g3_b),
        stn_fc1_W, row(stn_fc1_b), stn_fc2_W, row(stn_fc2_b),
        stn_fc3_W, row(stn_fc3_b),
        c1_Wr, c1_Wn, row(c1_b),
        c2_Wr, c2_Wn, row(c2_b),
        c3_Wr, c3_Wn, row(c3_b),
    )
    wspecs = [pl.BlockSpec(w.shape, lambda i: (0, 0)) for w in weights]

    h, t9 = pl.pallas_call(
        _fused_kernel,
        grid=(grid,),
        in_specs=[pl.BlockSpec((_BLOCK, 3), lambda i: (i, 0))] + wspecs,
        out_specs=[pl.BlockSpec((1, 1024), lambda i: (0, 0)),
                   pl.BlockSpec((1, 9), lambda i: (0, 0))],
        out_shape=[jax.ShapeDtypeStruct((1, 1024), jnp.float32),
                   jax.ShapeDtypeStruct((1, 9), jnp.float32)],
        scratch_shapes=[pltpu.VMEM((1, 1024), jnp.float32),
                        pltpu.VMEM((1, 1024), jnp.float32)],
    )(x, *weights)
    return h, t9.reshape(3, 3)


# speedup vs baseline: 3.8454x; 1.1543x over previous
import jax
import jax.numpy as jnp
from jax.experimental import pallas as pl
from jax.experimental.pallas import tpu as pltpu

_BLOCK = 4000
_NEG = -jnp.inf


def _mm(a, w):
    return jax.lax.dot_general(a, w, (((1,), (0,)), ((), ())),
                               preferred_element_type=jnp.float32)


def _mm16(a, w):
    return jax.lax.dot_general(a.astype(jnp.bfloat16),
                               w.astype(jnp.bfloat16),
                               (((1,), (0,)), ((), ())),
                               preferred_element_type=jnp.float32)


def _bulk_kernel(x_ref,
                 sWr1, sb1, sWr2, sb2, sWr3,
                 cWr1, cb1, cWr2, cb2, cWr3,
                 smax_out, cmax_out,
                 smax, cmax):
    i = pl.program_id(0)
    nsteps = pl.num_programs(0)
    xb = x_ref[...]

    hs = jnp.maximum(_mm(xb, sWr1[...]) + sb1[...], 0.0)
    hs = jnp.maximum(_mm(hs, sWr2[...]) + sb2[...], 0.0)
    bs = jnp.max(_mm16(hs, sWr3[...]), axis=0, keepdims=True)
    hc = jnp.maximum(_mm(xb, cWr1[...]) + cb1[...], 0.0)
    hc = jnp.maximum(_mm(hc, cWr2[...]) + cb2[...], 0.0)
    bc = jnp.max(_mm16(hc, cWr3[...]), axis=0, keepdims=True)

    @pl.when(i == 0)
    def _init():
        smax[...] = jnp.full((1, 1024), _NEG, jnp.float32)
        cmax[...] = jnp.full((1, 1024), _NEG, jnp.float32)

    smax[...] = jnp.maximum(smax[...], bs)
    cmax[...] = jnp.maximum(cmax[...], bc)

    @pl.when(i == nsteps - 1)
    def _out():
        smax_out[...] = smax[...]
        cmax_out[...] = cmax[...]


def _tail_kernel(x8_ref,
                 sWr1, sWn1, sb1, sWr2, sWn2, sb2, sWr3, sWn3, sb3,
                 fc1W, fc1b, fc2W, fc2b, fc3W, fc3b,
                 cWr1, cWn1, cb1, cWr2, cWn2, cb2, cWr3, cWn3, cb3,
                 smax_ref, cmax_ref,
                 h_out, t9_out):
    x8 = x8_ref[...]
    rows = jax.lax.broadcasted_iota(jnp.int32, (8, 1), 0)
    sel = (rows == 1).astype(jnp.float32)
    keep = rows < 2

    def gconv(h, wr, wn):
        return _mm(h, wr[...]) + sel * _mm(h[0:1, :], wn[...])

    e = jnp.maximum(gconv(x8, sWr1, sWn1) + sb1[...], 0.0)
    e = jnp.maximum(gconv(e, sWr2, sWn2) + sb2[...], 0.0)
    e = gconv(e, sWr3, sWn3)
    es = jnp.max(jnp.where(keep, e, _NEG), axis=0, keepdims=True)
    e = jnp.maximum(gconv(x8, cWr1, cWn1) + cb1[...], 0.0)
    e = jnp.maximum(gconv(e, cWr2, cWn2) + cb2[...], 0.0)
    e = gconv(e, cWr3, cWn3)
    ec = jnp.max(jnp.where(keep, e, _NEG), axis=0, keepdims=True)

    h_out[...] = jnp.maximum(cmax_ref[...], ec) + cb3[...]
    s = jnp.maximum(jnp.maximum(smax_ref[...], es) + sb3[...], 0.0)
    t = jnp.maximum(_mm(s, fc1W[...]) + fc1b[...], 0.0)
    t = jnp.maximum(_mm(t, fc2W[...]) + fc2b[...], 0.0)
    t9 = _mm(t, fc3W[...]) + fc3b[...]
    col = jax.lax.broadcasted_iota(jnp.int32, (1, 9), 1)
    t9_out[...] = t9 + (col % 4 == 0).astype(jnp.float32)


def kernel(x, stn_g1_Wr, stn_g1_Wn, stn_g1_b, stn_g2_Wr, stn_g2_Wn, stn_g2_b,
           stn_g3_Wr, stn_g3_Wn, stn_g3_b, stn_fc1_W, stn_fc1_b,
           stn_fc2_W, stn_fc2_b, stn_fc3_W, stn_fc3_b,
           c1_Wr, c1_Wn, c1_b, c2_Wr, c2_Wn, c2_b, c3_Wr, c3_Wn, c3_b):
    n = x.shape[0]
    grid = n // _BLOCK
    assert grid * _BLOCK == n

    x8 = x[0:8]
    x_bulk = x.at[1].set(x[0])

    row = lambda v: v.reshape(1, -1)
    bulk_w = (
        stn_g1_Wr, row(stn_g1_b), stn_g2_Wr, row(stn_g2_b), stn_g3_Wr,
        c1_Wr, row(c1_b), c2_Wr, row(c2_b), c3_Wr,
    )
    bspecs = [pl.BlockSpec(w.shape, lambda i: (0, 0)) for w in bulk_w]

    smax, cmax = pl.pallas_call(
        _bulk_kernel,
        grid=(grid,),
        in_specs=[pl.BlockSpec((_BLOCK, 3), lambda i: (i, 0))] + bspecs,
        out_specs=[pl.BlockSpec((1, 1024), lambda i: (0, 0)),
                   pl.BlockSpec((1, 1024), lambda i: (0, 0))],
        out_shape=[jax.ShapeDtypeStruct((1, 1024), jnp.float32),
                   jax.ShapeDtypeStruct((1, 1024), jnp.float32)],
        scratch_shapes=[pltpu.VMEM((1, 1024), jnp.float32),
                        pltpu.VMEM((1, 1024), jnp.float32)],
    )(x_bulk, *bulk_w)

    tail_in = (
        x8,
        stn_g1_Wr, stn_g1_Wn, row(stn_g1_b),
        stn_g2_Wr, stn_g2_Wn, row(stn_g2_b),
        stn_g3_Wr, stn_g3_Wn, row(stn_g3_b),
        stn_fc1_W, row(stn_fc1_b), stn_fc2_W, row(stn_fc2_b),
        stn_fc3_W, row(stn_fc3_b),
        c1_Wr, c1_Wn, row(c1_b),
        c2_Wr, c2_Wn, row(c2_b),
        c3_Wr, c3_Wn, row(c3_b),
        smax, cmax,
    )
    h, t9 = pl.pallas_call(
        _tail_kernel,
        out_shape=[jax.ShapeDtypeStruct((1, 1024), jnp.float32),
                   jax.ShapeDtypeStruct((1, 9), jnp.float32)],
    )(*tail_in)
    return h, t9.reshape(3, 3)
